# baseline (speedup 1.0000x reference)
import functools

import jax
import jax.numpy as jnp
from jax import lax
from jax.experimental import pallas as pl
from jax.experimental.pallas import tpu as pltpu
from jax.experimental.pallas import tpu_sc as plsc

M, N, DIM = 128, 64, 256
MN = M * N
ALPHA = 0.3
SIGMA = 64.0
EPS = 1e-6
L = 16
NC, NS = 2, 16
NW = NC * NS
ROWS = MN // NW
CH = DIM // L

_mesh = plsc.VectorSubcoreMesh(core_axis_name="c", subcore_axis_name="s")


_SCRATCH = [
    pltpu.VMEM((ROWS, DIM), jnp.float32),
    pltpu.VMEM((DIM,), jnp.float32),
    pltpu.VMEM((DIM,), jnp.float32),
    pltpu.VMEM((L,), jnp.float32),
    pltpu.VMEM((L,), jnp.int32),
    pltpu.VMEM((2 * L,), jnp.float32),
    pltpu.VMEM((NS, L), jnp.float32),
    pltpu.VMEM((NS, L), jnp.int32),
]


def _som_body(x_hbm, w_hbm, loc_hbm, out_hbm, exv_hbm, exi_hbm,
                wslab, xv, xe, recv, reci, lrb, av, ai):
    del loc_hbm
    c = lax.axis_index("c")
    s = lax.axis_index("s")
    wid = c * NS + s
    pwid = (1 - c) * NS + s

    pltpu.sync_copy(x_hbm, xv)
    for ci in range(CH):
        xe[pl.ds(ci * L, L)] = xv[pl.ds(ci * L, L)] + EPS

    def scan_slab(base_row, carry):
        def row_body(r, cr):
            bestv, besti = cr
            acc = jnp.zeros((L,), jnp.float32)
            for ci in range(CH):
                d = xe[pl.ds(ci * L, L)] - wslab[r, pl.ds(ci * L, L)]
                acc = acc + d * d
            dist = jnp.sum(acc)
            row = base_row + r
            better = (dist < bestv) | ((dist == bestv) & (row < besti))
            return (jnp.where(better, dist, bestv),
                    jnp.where(better, row, besti))
        return lax.fori_loop(0, ROWS, row_body, carry)

    pltpu.sync_copy(w_hbm.at[pl.ds(pwid * ROWS, ROWS)], wslab)
    carry = scan_slab(pwid * ROWS, (jnp.float32(3.0e38), jnp.int32(0)))
    pltpu.sync_copy(w_hbm.at[pl.ds(wid * ROWS, ROWS)], wslab)
    bestv, besti = scan_slab(wid * ROWS, carry)

    recv[:] = jnp.full((L,), bestv, jnp.float32)
    reci[:] = jnp.full((L,), besti, jnp.int32)
    pltpu.sync_copy(recv, exv_hbm.at[c, s])
    pltpu.sync_copy(reci, exi_hbm.at[c, s])
    plsc.subcore_barrier()
    pltpu.sync_copy(exv_hbm.at[c], av)
    pltpu.sync_copy(exi_hbm.at[c], ai)
    m = av[0]
    mi = ai[0]
    for t in range(1, NS):
        v = av[t]
        i = ai[t]
        lt = (v < m) | ((v == m) & (i < mi))
        m = jnp.where(lt, v, m)
        mi = jnp.where(lt, i, mi)
    bx = mi % M
    by = mi // M

    neg_inv_s2 = jnp.float32(-1.0 / (SIGMA * SIGMA))
    iota = lax.iota(jnp.int32, L)
    base_row = wid * ROWS

    def grp_body(g, _):
        rows = base_row + g * L + iota
        dx = (rows % M - bx).astype(jnp.float32)
        dy = (rows // M - by).astype(jnp.float32)
        lrb[pl.ds(0, L)] = ALPHA * jnp.exp((dx * dx + dy * dy) * neg_inv_s2)

        def row_body(l, __):
            lr = lrb[pl.ds(l, L)][0]
            r = g * L + l
            for ci in range(CH):
                w16 = wslab[r, pl.ds(ci * L, L)]
                wslab[r, pl.ds(ci * L, L)] = (
                    w16 + lr * (xv[pl.ds(ci * L, L)] - w16))
            return 0

        return lax.fori_loop(0, L, row_body, 0)

    lax.fori_loop(0, ROWS // L, grp_body, 0)
    pltpu.sync_copy(wslab, out_hbm.at[pl.ds(base_row, ROWS)])


_som_update = pl.kernel(
    _som_body,
    out_type=(
        jax.ShapeDtypeStruct((MN, DIM), jnp.float32),
        jax.ShapeDtypeStruct((NC, NS, L), jnp.float32),
        jax.ShapeDtypeStruct((NC, NS, L), jnp.int32),
    ),
    mesh=_mesh,
    scratch_types=_SCRATCH,
    compiler_params=pltpu.CompilerParams(needs_layout_passes=False),
)


def kernel(x, weights, locations):
    return _som_update(x, weights, locations)[0]
